# baseline (speedup 1.0000x reference)
import functools

import jax
import jax.numpy as jnp
from jax import lax
from jax.experimental import pallas as pl
from jax.experimental.pallas import tpu as pltpu
from jax.experimental.pallas import tpu_sc as plsc

N = 10000
E = 160000
D = 256
NP = 10240
RT = 640
ET = 172032
EB = 84
K = 128
IC = 28
NCH = EB // IC
TRASH = N

_MESH = plsc.VectorSubcoreMesh(core_axis_name="c", subcore_axis_name="s")


def _host_perms():
    try:
        import numpy as _np
        cpu = jax.devices("cpu")[0]
        with jax.default_device(cpu):
            pk1, pk2 = jax.random.split(jax.random.key(1))
            p1 = _np.asarray(jax.random.permutation(pk1, N)).astype(_np.int32)
            p2 = _np.asarray(jax.random.permutation(pk2, N)).astype(_np.int32)
        return p1, p2
    except Exception:
        return None


_PERMS = _host_perms()


def _deg_body(srcs_f, dsts_f, perm_a, perm_b, ones_h, deg_out, psrc_out,
              deg_sh, degp_sh, src_v, dst_v, ps_v, pd_v, ones_v, deg_v,
              asem, psem):
    g = lax.axis_index("c")
    t = lax.axis_index("s")
    r0 = t * RT

    for i in range(RT // 16):
        deg_v[pl.ds(16 * i, 16)] = jnp.zeros((16,), jnp.float32)
    pltpu.sync_copy(deg_v, deg_sh.at[pl.ds(r0, RT)])
    pltpu.sync_copy(deg_v, degp_sh.at[pl.ds(r0, RT)])
    pltpu.sync_copy(ones_h, ones_v)
    pltpu.sync_copy(srcs_f.at[g, t], src_v)
    pltpu.sync_copy(dsts_f.at[g, t], dst_v)
    plsc.subcore_barrier()

    @pl.loop(0, EB)
    def _fire(j):
        pltpu.async_copy(ones_v, deg_sh.at[dst_v.at[j]], asem, add=True)

    @pl.when(g == 0)
    def _fire_pa():
        @pl.loop(0, EB)
        def _fa(j):
            pltpu.async_copy(perm_a.at[dst_v.at[j]], pd_v.at[j], psem)
            pltpu.async_copy(perm_a.at[src_v.at[j]], ps_v.at[j], psem)

    @pl.when(g == 1)
    def _fire_pb():
        @pl.loop(0, EB)
        def _fb(j):
            pltpu.async_copy(perm_b.at[dst_v.at[j]], pd_v.at[j], psem)
            pltpu.async_copy(perm_b.at[src_v.at[j]], ps_v.at[j], psem)

    @pl.loop(0, 2 * EB)
    def _draing(j):
        pltpu.make_async_copy(perm_a.at[dst_v.at[0]], pd_v.at[0],
                              psem).wait()

    @pl.loop(0, EB)
    def _firep(j):
        pltpu.async_copy(ones_v, degp_sh.at[pd_v.at[j]], asem, add=True)

    pltpu.sync_copy(ps_v, psrc_out.at[g, t])

    @pl.loop(0, 2 * EB)
    def _drain(j):
        pltpu.make_async_copy(ones_v, deg_sh.at[dst_v.at[0]], asem).wait()

    plsc.subcore_barrier()
    pltpu.sync_copy(deg_sh.at[pl.ds(r0, RT)], deg_out.at[g, 0, pl.ds(r0, RT)])
    pltpu.sync_copy(degp_sh.at[pl.ds(r0, RT)], deg_out.at[g, 1, pl.ds(r0, RT)])


_deg_kernel = functools.partial(
    pl.kernel,
    out_type=[
        jax.ShapeDtypeStruct((2, 2, NP), jnp.float32),
        jax.ShapeDtypeStruct((2, 16, EB, K), jnp.int32),
    ],
    mesh=_MESH,
    scratch_types=[
        pltpu.VMEM_SHARED((NP,), jnp.float32),
        pltpu.VMEM_SHARED((NP,), jnp.float32),
        pltpu.VMEM((EB, K), jnp.int32),
        pltpu.VMEM((EB, K), jnp.int32),
        pltpu.VMEM((EB, K), jnp.int32),
        pltpu.VMEM((EB, K), jnp.int32),
        pltpu.VMEM((K,), jnp.float32),
        pltpu.VMEM((RT,), jnp.float32),
        pltpu.SemaphoreType.DMA,
        pltpu.SemaphoreType.DMA,
    ],
)(_deg_body)


def _tbl_body(x_ref, w_ref, deg_ref, o_ref):
    xwb = jnp.dot(x_ref[...], w_ref[0], preferred_element_type=jnp.float32)
    dgb = deg_ref[0, 0]
    dpb = deg_ref[0, 1]
    dis = jnp.where(dgb > 0, lax.rsqrt(dgb), 0.0)[:, None]
    ee = jnp.where(dpb > 0, lax.rsqrt(dpb), 0.0)[:, None]
    for c in range(2):
        o_ref[0, 0, c] = xwb[:, c * K:(c + 1) * K] * dis
        o_ref[0, 1, c] = xwb[:, c * K:(c + 1) * K] * ee


def _tbl_tc(x_pad, w_stacked, deg):
    rb = 1280
    nb = NP // rb
    return pl.pallas_call(
        _tbl_body,
        grid=(2, nb),
        in_specs=[
            pl.BlockSpec((rb, D), lambda g, i: (i, 0)),
            pl.BlockSpec((1, D, D), lambda g, i: (g, 0, 0)),
            pl.BlockSpec((1, 2, rb), lambda g, i: (g, 0, i)),
        ],
        out_specs=pl.BlockSpec((1, 2, 2, rb, K), lambda g, i: (g, 0, 0, i, 0)),
        out_shape=jax.ShapeDtypeStruct((2, 2, 2, NP, K), jnp.float32),
    )(x_pad, w_stacked, deg)


def _spmm_body(tbl, srcs, psrcs, dsts, zeros_h,
               acc_out,
               acc_sh, gi0, gi1, di0, di1, buf0, buf1, gsem, asem, rsem):
    g = lax.axis_index("c")
    t = lax.axis_index("s")
    r0 = t * RT
    bufs = (buf0, buf1)
    gidx = (gi0, gi1)
    didx = (di0, di1)

    for tt in range(2):
        idx_h = srcs if tt == 0 else psrcs
        for c in range(2):
            tref = tbl.at[g, tt, c]
            pltpu.sync_copy(zeros_h, acc_sh.at[pl.ds(r0, RT)])
            plsc.subcore_barrier()

            for q in range(NCH):
                qb = q % 2
                if q == 0:
                    pltpu.sync_copy(idx_h.at[g, t, 0], gidx[0])
                    pltpu.sync_copy(dsts.at[g, t, 0], didx[0])
                else:
                    pltpu.make_async_copy(
                        idx_h.at[g, t, q], gidx[qb], rsem).wait()
                    pltpu.make_async_copy(
                        dsts.at[g, t, q], didx[qb], rsem).wait()
                if q + 1 < NCH:
                    nq = (q + 1) % 2
                    pltpu.async_copy(idx_h.at[g, t, q + 1], gidx[nq], rsem)
                    pltpu.async_copy(dsts.at[g, t, q + 1], didx[nq], rsem)
                gi = gidx[qb]
                di = didx[qb]

                def gstart(j, b, gi=gi):
                    pltpu.async_copy(tref.at[gi.at[j]], bufs[b], gsem)

                def gwait():
                    pltpu.make_async_copy(tref.at[pl.ds(0, K)], buf0,
                                          gsem).wait()

                def astart(j, b, di=di):
                    pltpu.async_copy(bufs[b], acc_sh.at[di.at[j]], asem,
                                     add=True)

                def awaitd():
                    pltpu.make_async_copy(tref.at[pl.ds(0, K)], buf0,
                                          asem).wait()

                gstart(0, 0)
                gwait()
                gstart(1, 1)
                astart(0, 0)

                @pl.loop(0, (IC - 2) // 2, unroll=13)
                def _pipe(j2):
                    j = 2 * j2 + 1
                    gwait()
                    astart(j, 1)
                    awaitd()
                    gstart(j + 1, 0)
                    gwait()
                    astart(j + 1, 0)
                    awaitd()
                    gstart(j + 2, 1)

                awaitd()
                gwait()
                astart(IC - 1, 1)
                awaitd()

            plsc.subcore_barrier()
            pltpu.sync_copy(acc_sh.at[pl.ds(r0, RT)],
                            acc_out.at[g, tt, c, pl.ds(r0, RT)])


_spmm_kernel = functools.partial(
    pl.kernel,
    out_type=[jax.ShapeDtypeStruct((2, 2, 2, NP, K), jnp.float32)],
    mesh=_MESH,
    scratch_types=[
        pltpu.VMEM_SHARED((NP, K), jnp.float32),
        pltpu.VMEM((IC, K), jnp.int32),
        pltpu.VMEM((IC, K), jnp.int32),
        pltpu.VMEM((IC, K), jnp.int32),
        pltpu.VMEM((IC, K), jnp.int32),
        pltpu.VMEM((K, K), jnp.float32),
        pltpu.VMEM((K, K), jnp.float32),
        pltpu.SemaphoreType.DMA,
        pltpu.SemaphoreType.DMA,
        pltpu.SemaphoreType.DMA,
    ],
)(_spmm_body)


def _fin_body(acc_ref, deg_ref, bias_ref, a_ref, pos_ref, neg_ref, sum_ref):
    g = pl.program_id(0)
    c = pl.program_id(1)
    i = pl.program_id(2)
    dgb = deg_ref[0, 0, :, 0]
    dis = jnp.where(dgb > 0, lax.rsqrt(dgb), 0.0)[:, None]
    a = a_ref[g, 0]
    b = bias_ref[g, c][None, :]

    vp = acc_ref[0, 0, 0] * dis + b
    pos = jnp.where(vp > 0, vp, a * vp)
    pos_ref[0] = pos
    vn = acc_ref[0, 1, 0] * dis + b
    neg_ref[0] = jnp.where(vn > 0, vn, a * vn)

    part = jnp.sum(pos, axis=0, keepdims=True) * jnp.float32(1.0 / N)

    @pl.when(i == 0)
    def _init():
        sum_ref[0, 0] = part

    @pl.when(i > 0)
    def _accum():
        sum_ref[0, 0] = sum_ref[0, 0] + part


def _fin_tc(acc, deg, bias, a_st):
    rb = 400
    nb = N // rb
    pos, neg, sums = pl.pallas_call(
        _fin_body,
        grid=(2, 2, nb),
        in_specs=[
            pl.BlockSpec((1, 2, 1, rb, K), lambda g, c, i: (g, 0, c, i, 0)),
            pl.BlockSpec((1, 1, rb, 1), lambda g, c, i: (g, 0, i, 0)),
            pl.BlockSpec((2, 2, K), lambda g, c, i: (0, 0, 0)),
            pl.BlockSpec((2, 1), lambda g, c, i: (0, 0)),
        ],
        out_specs=[
            pl.BlockSpec((1, rb, K), lambda g, c, i: (g, i, c)),
            pl.BlockSpec((1, rb, K), lambda g, c, i: (g, i, c)),
            pl.BlockSpec((1, 1, 1, K), lambda g, c, i: (g, c, 0, 0)),
        ],
        out_shape=[
            jax.ShapeDtypeStruct((2, N, D), jnp.float32),
            jax.ShapeDtypeStruct((2, N, D), jnp.float32),
            jax.ShapeDtypeStruct((2, 2, 1, K), jnp.float32),
        ],
    )(acc, deg.reshape(2, 2, NP, 1), bias, a_st)
    return pos, neg, sums.reshape(2, D)


def _ext_edges(ei):
    src = ei[0].astype(jnp.int32)
    dst = ei[1].astype(jnp.int32)
    loop = jnp.arange(N, dtype=jnp.int32)
    pad = ET - (E + N)
    src_e = jnp.concatenate([src, loop, jnp.zeros((pad,), jnp.int32)])
    dst_e = jnp.concatenate([dst, loop, jnp.full((pad,), TRASH, jnp.int32)])
    return src_e, dst_e


def kernel(x, edge_index_a, edge_index_b, W1, b1, a1, W2, b2, a2):
    if _PERMS is not None:
        perm1 = jnp.asarray(_PERMS[0])
        perm2 = jnp.asarray(_PERMS[1])
    else:
        perm_key = jax.random.key(1)
        pk1, pk2 = jax.random.split(perm_key)
        perm1 = jax.random.permutation(pk1, N).astype(jnp.int32)
        perm2 = jax.random.permutation(pk2, N).astype(jnp.int32)
    ppad = jnp.arange(N, NP, dtype=jnp.int32)
    perm_st = jnp.stack([jnp.concatenate([perm1, ppad]),
                         jnp.concatenate([perm2, ppad])])

    src_a, dst_a = _ext_edges(edge_index_a)
    src_b, dst_b = _ext_edges(edge_index_b)

    srcs_f = jnp.stack([src_a, src_b]).reshape(2, 16, EB, K)
    dsts_f = jnp.stack([dst_a, dst_b]).reshape(2, 16, EB, K)
    srcs = srcs_f.reshape(2, 16, NCH, IC, K)
    dsts = dsts_f.reshape(2, 16, NCH, IC, K)
    ones = jnp.ones((K,), jnp.float32)
    zeros_h = jnp.zeros((RT, K), jnp.float32)

    deg, psrc_f = _deg_kernel(srcs_f, dsts_f, perm_st[0], perm_st[1], ones)
    psrcs = psrc_f.reshape(2, 16, NCH, IC, K)

    x_pad = jnp.pad(x, ((0, NP - N), (0, 0)))
    tbl = _tbl_tc(x_pad, jnp.stack([W1, W2]), deg)

    (acc,) = _spmm_kernel(tbl, srcs, psrcs, dsts, zeros_h)

    bias = jnp.stack([b1, b2]).reshape(2, 2, K)
    a_st = jnp.stack([a1, a2]).astype(jnp.float32).reshape(2, 1)
    pos, neg, sums = _fin_tc(acc, deg, bias, a_st)
    return pos, neg, sums


# speedup vs baseline: 1.1197x; 1.1197x over previous
import functools

import jax
import jax.numpy as jnp
from jax import lax
from jax.experimental import pallas as pl
from jax.experimental.pallas import tpu as pltpu
from jax.experimental.pallas import tpu_sc as plsc

N = 10000
E = 160000
D = 256
NP = 10240
RT = 640
ET = 172032
EB = 84
K = 128
IC = 28
NCH = EB // IC
TRASH = N

_MESH = plsc.VectorSubcoreMesh(core_axis_name="c", subcore_axis_name="s")


def _host_perms():
    try:
        import numpy as _np
        cpu = jax.devices("cpu")[0]
        with jax.default_device(cpu):
            pk1, pk2 = jax.random.split(jax.random.key(1))
            p1 = _np.asarray(jax.random.permutation(pk1, N)).astype(_np.int32)
            p2 = _np.asarray(jax.random.permutation(pk2, N)).astype(_np.int32)
        return p1, p2
    except Exception:
        return None


_PERMS = _host_perms()


def _deg_body(srcs_f, dsts_f, perm_a, perm_b, ones_h, deg_out, psrc_out,
              deg_sh, degp_sh, src_v, dst_v, ps_v, pd_v, ones_v, deg_v,
              asem, psem):
    g = lax.axis_index("c")
    t = lax.axis_index("s")
    r0 = t * RT

    for i in range(RT // 16):
        deg_v[pl.ds(16 * i, 16)] = jnp.zeros((16,), jnp.float32)
    pltpu.sync_copy(deg_v, deg_sh.at[pl.ds(r0, RT)])
    pltpu.sync_copy(deg_v, degp_sh.at[pl.ds(r0, RT)])
    pltpu.sync_copy(ones_h, ones_v)
    pltpu.sync_copy(srcs_f.at[g, t], src_v)
    pltpu.sync_copy(dsts_f.at[g, t], dst_v)
    plsc.subcore_barrier()

    @pl.loop(0, EB)
    def _fire(j):
        pltpu.async_copy(ones_v, deg_sh.at[dst_v.at[j]], asem, add=True)

    @pl.when(g == 0)
    def _fire_pa():
        @pl.loop(0, EB)
        def _fa(j):
            pltpu.async_copy(perm_a.at[dst_v.at[j]], pd_v.at[j], psem)
            pltpu.async_copy(perm_a.at[src_v.at[j]], ps_v.at[j], psem)

    @pl.when(g == 1)
    def _fire_pb():
        @pl.loop(0, EB)
        def _fb(j):
            pltpu.async_copy(perm_b.at[dst_v.at[j]], pd_v.at[j], psem)
            pltpu.async_copy(perm_b.at[src_v.at[j]], ps_v.at[j], psem)

    @pl.loop(0, 2 * EB)
    def _draing(j):
        pltpu.make_async_copy(perm_a.at[dst_v.at[0]], pd_v.at[0],
                              psem).wait()

    @pl.loop(0, EB)
    def _firep(j):
        pltpu.async_copy(ones_v, degp_sh.at[pd_v.at[j]], asem, add=True)

    pltpu.sync_copy(ps_v, psrc_out.at[g, t])

    @pl.loop(0, 2 * EB)
    def _drain(j):
        pltpu.make_async_copy(ones_v, deg_sh.at[dst_v.at[0]], asem).wait()

    plsc.subcore_barrier()
    pltpu.sync_copy(deg_sh.at[pl.ds(r0, RT)], deg_out.at[g, 0, pl.ds(r0, RT)])
    pltpu.sync_copy(degp_sh.at[pl.ds(r0, RT)], deg_out.at[g, 1, pl.ds(r0, RT)])


_deg_kernel = functools.partial(
    pl.kernel,
    out_type=[
        jax.ShapeDtypeStruct((2, 2, NP), jnp.float32),
        jax.ShapeDtypeStruct((2, 16, EB, K), jnp.int32),
    ],
    mesh=_MESH,
    scratch_types=[
        pltpu.VMEM_SHARED((NP,), jnp.float32),
        pltpu.VMEM_SHARED((NP,), jnp.float32),
        pltpu.VMEM((EB, K), jnp.int32),
        pltpu.VMEM((EB, K), jnp.int32),
        pltpu.VMEM((EB, K), jnp.int32),
        pltpu.VMEM((EB, K), jnp.int32),
        pltpu.VMEM((K,), jnp.float32),
        pltpu.VMEM((RT,), jnp.float32),
        pltpu.SemaphoreType.DMA,
        pltpu.SemaphoreType.DMA,
    ],
)(_deg_body)


def _tbl_body(x_ref, w_ref, deg_ref, o_ref):
    xwb = jnp.dot(x_ref[...], w_ref[0], preferred_element_type=jnp.float32)
    dgb = deg_ref[0, 0]
    dpb = deg_ref[0, 1]
    dis = jnp.where(dgb > 0, lax.rsqrt(dgb), 0.0)[:, None]
    ee = jnp.where(dpb > 0, lax.rsqrt(dpb), 0.0)[:, None]
    for c in range(2):
        o_ref[0, 0, c] = xwb[:, c * K:(c + 1) * K] * dis
        o_ref[0, 1, c] = xwb[:, c * K:(c + 1) * K] * ee


def _tbl_tc(x_pad, w_stacked, deg):
    rb = 1280
    nb = NP // rb
    return pl.pallas_call(
        _tbl_body,
        grid=(2, nb),
        in_specs=[
            pl.BlockSpec((rb, D), lambda g, i: (i, 0)),
            pl.BlockSpec((1, D, D), lambda g, i: (g, 0, 0)),
            pl.BlockSpec((1, 2, rb), lambda g, i: (g, 0, i)),
        ],
        out_specs=pl.BlockSpec((1, 2, 2, rb, K), lambda g, i: (g, 0, 0, i, 0)),
        out_shape=jax.ShapeDtypeStruct((2, 2, 2, NP, K), jnp.float32),
    )(x_pad, w_stacked, deg)


def _spmm_body(tbl, srcs, psrcs, srcs1, psrcs1, dsts, zeros_h,
               acc_out,
               acc_sh, g1a, g1b, di0, di1, buf0, buf1, gsem, asem, rsem):
    g = lax.axis_index("c")
    t = lax.axis_index("s")
    r0 = t * RT
    bufs = (buf0, buf1)
    gidx = (g1a, g1b)
    didx = (di0, di1)
    SEG = 32
    NSEG = K // SEG

    for tt in range(2):
        idx_h = srcs1 if tt == 0 else psrcs1
        for c in range(2):
            tref = tbl.at[g, tt, c]
            pltpu.sync_copy(zeros_h, acc_sh.at[pl.ds(r0, RT)])
            plsc.subcore_barrier()

            for q in range(NCH):
                qb = q % 2
                if q == 0:
                    pltpu.sync_copy(idx_h.at[g, t, 0, 0], gidx[0])
                    pltpu.sync_copy(dsts.at[g, t, 0], didx[0])
                else:
                    pltpu.make_async_copy(
                        idx_h.at[g, t, q, 0], gidx[qb], rsem).wait()
                    pltpu.make_async_copy(
                        dsts.at[g, t, q], didx[qb], rsem).wait()
                if q + 1 < NCH:
                    nq = (q + 1) % 2
                    pltpu.async_copy(idx_h.at[g, t, q + 1, 0], gidx[nq], rsem)
                    pltpu.async_copy(dsts.at[g, t, q + 1], didx[nq], rsem)
                gi = gidx[qb]
                di = didx[qb]

                def gstart4(j, b, gi=gi):
                    for sg in range(NSEG):
                        pltpu.async_copy(
                            tref.at[gi.at[pl.ds(j * K + sg * SEG, SEG)]],
                            bufs[b].at[pl.ds(sg * SEG, SEG)], gsem)

                def gwait4():
                    for sg in range(NSEG):
                        pltpu.make_async_copy(
                            tref.at[pl.ds(0, SEG)],
                            buf0.at[pl.ds(0, SEG)], gsem).wait()

                def astart(j, b, di=di):
                    pltpu.async_copy(bufs[b], acc_sh.at[di.at[j]], asem,
                                     add=True)

                def awaitd():
                    pl_sem_wait_add()

                def pl_sem_wait_add():
                    pltpu.make_async_copy(tref.at[pl.ds(0, K)], buf0,
                                          asem).wait()

                gstart4(0, 0)
                gstart4(1, 1)
                gwait4()
                astart(0, 0)

                @pl.loop(0, (IC - 2) // 2, unroll=4)
                def _pipe(j2):
                    j = 2 * j2 + 1
                    awaitd()
                    gstart4(j + 1, 0)
                    gwait4()
                    astart(j, 1)
                    awaitd()
                    gstart4(j + 2, 1)
                    gwait4()
                    astart(j + 1, 0)

                awaitd()
                gwait4()
                astart(IC - 1, 1)
                awaitd()

            plsc.subcore_barrier()
            pltpu.sync_copy(acc_sh.at[pl.ds(r0, RT)],
                            acc_out.at[g, tt, c, pl.ds(r0, RT)])


_spmm_kernel = functools.partial(
    pl.kernel,
    out_type=[jax.ShapeDtypeStruct((2, 2, 2, NP, K), jnp.float32)],
    mesh=_MESH,
    scratch_types=[
        pltpu.VMEM_SHARED((NP, K), jnp.float32),
        pltpu.VMEM((IC * K,), jnp.int32),
        pltpu.VMEM((IC * K,), jnp.int32),
        pltpu.VMEM((IC, K), jnp.int32),
        pltpu.VMEM((IC, K), jnp.int32),
        pltpu.VMEM((K, K), jnp.float32),
        pltpu.VMEM((K, K), jnp.float32),
        pltpu.SemaphoreType.DMA,
        pltpu.SemaphoreType.DMA,
        pltpu.SemaphoreType.DMA,
    ],
)(_spmm_body)


def _fin_body(acc_ref, deg_ref, bias_ref, a_ref, pos_ref, neg_ref, sum_ref):
    g = pl.program_id(0)
    c = pl.program_id(1)
    i = pl.program_id(2)
    dgb = deg_ref[0, 0, :, 0]
    dis = jnp.where(dgb > 0, lax.rsqrt(dgb), 0.0)[:, None]
    a = a_ref[g, 0]
    b = bias_ref[g, c][None, :]

    vp = acc_ref[0, 0, 0] * dis + b
    pos = jnp.where(vp > 0, vp, a * vp)
    pos_ref[0] = pos
    vn = acc_ref[0, 1, 0] * dis + b
    neg_ref[0] = jnp.where(vn > 0, vn, a * vn)

    part = jnp.sum(pos, axis=0, keepdims=True) * jnp.float32(1.0 / N)

    @pl.when(i == 0)
    def _init():
        sum_ref[0, 0] = part

    @pl.when(i > 0)
    def _accum():
        sum_ref[0, 0] = sum_ref[0, 0] + part


def _fin_tc(acc, deg, bias, a_st):
    rb = 400
    nb = N // rb
    pos, neg, sums = pl.pallas_call(
        _fin_body,
        grid=(2, 2, nb),
        in_specs=[
            pl.BlockSpec((1, 2, 1, rb, K), lambda g, c, i: (g, 0, c, i, 0)),
            pl.BlockSpec((1, 1, rb, 1), lambda g, c, i: (g, 0, i, 0)),
            pl.BlockSpec((2, 2, K), lambda g, c, i: (0, 0, 0)),
            pl.BlockSpec((2, 1), lambda g, c, i: (0, 0)),
        ],
        out_specs=[
            pl.BlockSpec((1, rb, K), lambda g, c, i: (g, i, c)),
            pl.BlockSpec((1, rb, K), lambda g, c, i: (g, i, c)),
            pl.BlockSpec((1, 1, 1, K), lambda g, c, i: (g, c, 0, 0)),
        ],
        out_shape=[
            jax.ShapeDtypeStruct((2, N, D), jnp.float32),
            jax.ShapeDtypeStruct((2, N, D), jnp.float32),
            jax.ShapeDtypeStruct((2, 2, 1, K), jnp.float32),
        ],
    )(acc, deg.reshape(2, 2, NP, 1), bias, a_st)
    return pos, neg, sums.reshape(2, D)


def _ext_edges(ei):
    src = ei[0].astype(jnp.int32)
    dst = ei[1].astype(jnp.int32)
    loop = jnp.arange(N, dtype=jnp.int32)
    pad = ET - (E + N)
    src_e = jnp.concatenate([src, loop, jnp.zeros((pad,), jnp.int32)])
    dst_e = jnp.concatenate([dst, loop, jnp.full((pad,), TRASH, jnp.int32)])
    return src_e, dst_e


def kernel(x, edge_index_a, edge_index_b, W1, b1, a1, W2, b2, a2):
    if _PERMS is not None:
        perm1 = jnp.asarray(_PERMS[0])
        perm2 = jnp.asarray(_PERMS[1])
    else:
        perm_key = jax.random.key(1)
        pk1, pk2 = jax.random.split(perm_key)
        perm1 = jax.random.permutation(pk1, N).astype(jnp.int32)
        perm2 = jax.random.permutation(pk2, N).astype(jnp.int32)
    ppad = jnp.arange(N, NP, dtype=jnp.int32)
    perm_st = jnp.stack([jnp.concatenate([perm1, ppad]),
                         jnp.concatenate([perm2, ppad])])

    src_a, dst_a = _ext_edges(edge_index_a)
    src_b, dst_b = _ext_edges(edge_index_b)

    srcs_f = jnp.stack([src_a, src_b]).reshape(2, 16, EB, K)
    dsts_f = jnp.stack([dst_a, dst_b]).reshape(2, 16, EB, K)
    srcs = srcs_f.reshape(2, 16, NCH, IC, K)
    dsts = dsts_f.reshape(2, 16, NCH, IC, K)
    srcs1 = srcs_f.reshape(2, 16, NCH, 1, IC * K)
    psrcs1_shape = (2, 16, NCH, 1, IC * K)
    ones = jnp.ones((K,), jnp.float32)
    zeros_h = jnp.zeros((RT, K), jnp.float32)

    deg, psrc_f = _deg_kernel(srcs_f, dsts_f, perm_st[0], perm_st[1], ones)
    psrcs = psrc_f.reshape(2, 16, NCH, IC, K)

    x_pad = jnp.pad(x, ((0, NP - N), (0, 0)))
    tbl = _tbl_tc(x_pad, jnp.stack([W1, W2]), deg)

    psrcs1 = psrc_f.reshape(psrcs1_shape)
    (acc,) = _spmm_kernel(tbl, srcs, psrcs, srcs1, psrcs1, dsts, zeros_h)

    bias = jnp.stack([b1, b2]).reshape(2, 2, K)
    a_st = jnp.stack([a1, a2]).astype(jnp.float32).reshape(2, 1)
    pos, neg, sums = _fin_tc(acc, deg, bias, a_st)
    return pos, neg, sums
